# baseline (speedup 1.0000x reference)
import functools

import jax
import jax.numpy as jnp
from jax import lax
from jax.experimental import pallas as pl
from jax.experimental.pallas import tpu as pltpu
from jax.experimental.pallas import tpu_sc as plsc

D_MODEL = 2048
SEQ_LEN = 4096
BATCH = 4

SC_BATCH = 2
TC_BATCH = BATCH - SC_BATCH

NC, NS, L = 2, 16, 16
NW = NC * NS
SEQ_PER_W = SEQ_LEN // NW

CHUNK = 8
CHUNK_ELEMS = CHUNK * D_MODEL
N_SEQ_CHUNKS = SEQ_PER_W // CHUNK
COL_ITERS = 8
COL_UNROLL = D_MODEL // (COL_ITERS * L)


def _sc_body(x_hbm, pos_hbm, out_hbm,
             xb0, xb1, pb0, pb1, ob0, ob1, sem_x, sem_p, sem_o):
    c = lax.axis_index("c")
    s = lax.axis_index("s")
    wid = s * NC + c
    seq0 = wid * SEQ_PER_W

    xbufs = (xb0, xb1)
    pbufs = (pb0, pb1)
    obufs = (ob0, ob1)

    def x_row(b, sc):
        return b * SEQ_LEN + seq0 + sc * CHUNK

    def p_row(sc):
        return seq0 + sc * CHUNK

    def start_x(b, sc, dst):
        pltpu.async_copy(x_hbm.at[pl.ds(x_row(b, sc), CHUNK)], dst, sem_x)

    def start_p(sc, dst):
        pltpu.async_copy(pos_hbm.at[pl.ds(p_row(sc), CHUNK)], dst, sem_p)

    def start_o(b, sc, src):
        pltpu.async_copy(src, out_hbm.at[pl.ds(x_row(b, sc), CHUNK)], sem_o)

    def wait_x(dst):
        pltpu.make_async_copy(x_hbm.at[pl.ds(0, CHUNK)], dst, sem_x).wait()

    def wait_p(dst):
        pltpu.make_async_copy(pos_hbm.at[pl.ds(0, CHUNK)], dst, sem_p).wait()

    def wait_o(src):
        pltpu.make_async_copy(src, out_hbm.at[pl.ds(0, CHUNK)], sem_o).wait()

    def add_chunk(xr, pr, outr):
        def body(i, acc):
            base = i * (COL_UNROLL * L)
            for r in range(CHUNK):
                for j in range(COL_UNROLL):
                    o = base + j * L
                    outr[r, pl.ds(o, L)] = xr[r, pl.ds(o, L)] + pr[r, pl.ds(o, L)]
            return acc

        lax.fori_loop(0, COL_ITERS, body, 0)

    start_p(0, pb0)
    start_x(0, 0, xb0)

    def sc_block(j, sc, pslot, last):
        pbuf = pbufs[pslot]
        for b in range(SC_BATCH):
            xbuf = xbufs[b % 2]
            obuf = obufs[b % 2]
            wait_x(xbuf)
            if b < SC_BATCH - 1:
                start_x(b + 1, sc, xbufs[(b + 1) % 2])
            elif not last:
                start_x(0, sc + 1, xbufs[0])
            else:
                @pl.when(j < (N_SEQ_CHUNKS // 2) - 1)
                def _():
                    start_x(0, sc + 1, xbufs[0])
            if b == 0:
                wait_p(pbuf)
                if not last:
                    start_p(sc + 1, pbufs[1 - pslot])
                else:
                    @pl.when(j < (N_SEQ_CHUNKS // 2) - 1)
                    def _():
                        start_p(sc + 1, pbufs[1 - pslot])
            if pslot == 0:
                @pl.when(j >= 1)
                def _():
                    wait_o(obuf)
            else:
                wait_o(obuf)
            add_chunk(xbuf, pbuf, obuf)
            start_o(b, sc, obuf)

    def loop_body(j, acc):
        sc_block(j, 2 * j, 0, last=False)
        sc_block(j, 2 * j + 1, 1, last=True)
        return acc

    lax.fori_loop(0, N_SEQ_CHUNKS // 2, loop_body, 0)

    wait_o(ob0)
    wait_o(ob1)


_sc_add = functools.partial(
    pl.kernel,
    out_type=jax.ShapeDtypeStruct((SC_BATCH * SEQ_LEN, D_MODEL), jnp.float32),
    mesh=plsc.VectorSubcoreMesh(core_axis_name="c", subcore_axis_name="s"),
    scratch_types=[
        pltpu.VMEM((CHUNK, D_MODEL), jnp.float32),
        pltpu.VMEM((CHUNK, D_MODEL), jnp.float32),
        pltpu.VMEM((CHUNK, D_MODEL), jnp.float32),
        pltpu.VMEM((CHUNK, D_MODEL), jnp.float32),
        pltpu.VMEM((CHUNK, D_MODEL), jnp.float32),
        pltpu.VMEM((CHUNK, D_MODEL), jnp.float32),
        pltpu.SemaphoreType.DMA,
        pltpu.SemaphoreType.DMA,
        pltpu.SemaphoreType.DMA,
    ],
    compiler_params=pltpu.CompilerParams(use_tc_tiling_on_sc=True),
)(_sc_body)


TC_BS = 512


def _tc_body(x_ref, p_ref, o_ref):
    o_ref[...] = x_ref[...] + p_ref[...]


_tc_add = pl.pallas_call(
    _tc_body,
    grid=(TC_BATCH, SEQ_LEN // TC_BS),
    in_specs=[
        pl.BlockSpec((1, TC_BS, D_MODEL), lambda b, i: (b + SC_BATCH, i, 0)),
        pl.BlockSpec((TC_BS, D_MODEL), lambda b, i: (i, 0)),
    ],
    out_specs=pl.BlockSpec((1, TC_BS, D_MODEL), lambda b, i: (b, i, 0)),
    out_shape=jax.ShapeDtypeStruct((TC_BATCH, SEQ_LEN, D_MODEL), jnp.float32),
)


@jax.jit
def kernel(x, pos_table):
    x2 = x.reshape(BATCH * SEQ_LEN, D_MODEL)
    out_sc = _sc_add(x2, pos_table)
    out_tc = _tc_add(x, pos_table)
    out = jnp.concatenate(
        [out_sc.reshape(SC_BATCH, SEQ_LEN, D_MODEL), out_tc], axis=0)
    return out


# speedup vs baseline: 1.2300x; 1.2300x over previous
import functools

import jax
import jax.numpy as jnp
from jax import lax
from jax.experimental import pallas as pl
from jax.experimental.pallas import tpu as pltpu
from jax.experimental.pallas import tpu_sc as plsc

D_MODEL = 2048
SEQ_LEN = 4096
BATCH = 4

NC, NS, L = 2, 16, 16
NW = NC * NS
SEQ_PER_W = SEQ_LEN // NW

CHUNK = 8
CHUNK_ELEMS = CHUNK * D_MODEL
N_SEQ_CHUNKS = SEQ_PER_W // CHUNK
COL_ITERS = 8
COL_UNROLL = D_MODEL // (COL_ITERS * L)
NXB = 4


def _sc_body(x_hbm, pos_hbm, out_hbm,
             xb0, xb1, xb2, xb3, pb0, pb1, sem_x, sem_p, sem_o):
    c = lax.axis_index("c")
    s = lax.axis_index("s")
    wid = s * NC + c
    seq0 = wid * SEQ_PER_W

    xbufs = (xb0, xb1, xb2, xb3)
    pbufs = (pb0, pb1)

    def x_row(b, sc):
        return b * SEQ_LEN + seq0 + sc * CHUNK

    def p_row(sc):
        return seq0 + sc * CHUNK

    def start_x(b, sc, dst):
        pltpu.async_copy(x_hbm.at[pl.ds(x_row(b, sc), CHUNK)], dst, sem_x)

    def start_p(sc, dst):
        pltpu.async_copy(pos_hbm.at[pl.ds(p_row(sc), CHUNK)], dst, sem_p)

    def start_o(b, sc, src):
        pltpu.async_copy(src, out_hbm.at[pl.ds(x_row(b, sc), CHUNK)], sem_o)

    def wait_x(dst):
        pltpu.make_async_copy(x_hbm.at[pl.ds(0, CHUNK)], dst, sem_x).wait()

    def wait_p(dst):
        pltpu.make_async_copy(pos_hbm.at[pl.ds(0, CHUNK)], dst, sem_p).wait()

    def wait_o(src):
        pltpu.make_async_copy(src, out_hbm.at[pl.ds(0, CHUNK)], sem_o).wait()

    def add_chunk(pbuf, xbuf):
        def body(i, acc):
            base = i * (COL_UNROLL * L)
            for r in range(CHUNK):
                for j in range(COL_UNROLL):
                    o = base + j * L
                    plsc.addupdate(xbuf.at[r, pl.ds(o, L)],
                                   pbuf[r, pl.ds(o, L)])
            return acc

        lax.fori_loop(0, COL_ITERS, body, 0)

    start_p(0, pb0)
    start_x(0, 0, xb0)
    start_x(1, 0, xb1)
    start_x(2, 0, xb2)

    def sc_block(j, sc, pslot, last):
        pbuf = pbufs[pslot]
        for b in range(BATCH):
            xbuf = xbufs[b]
            wait_x(xbuf)
            if b == 0:
                wait_p(pbuf)
                if not last:
                    start_p(sc + 1, pbufs[1 - pslot])
                else:
                    @pl.when(j < (N_SEQ_CHUNKS // 2) - 1)
                    def _():
                        start_p(sc + 1, pbufs[1 - pslot])
            add_chunk(pbuf, xbuf)
            start_o(b, sc, xbuf)
            nb, nsc = (3, sc) if b == 0 else (b - 1, sc + 1)
            prev_slot = xbufs[(b + 3) % 4]
            if pslot == 0 and b == 0:
                @pl.when(j >= 1)
                def _():
                    wait_o(prev_slot)
            else:
                wait_o(prev_slot)
            if pslot == 0 or b == 0:
                start_x(nb, nsc, prev_slot)
            else:
                @pl.when(j < (N_SEQ_CHUNKS // 2) - 1)
                def _():
                    start_x(nb, nsc, prev_slot)

    def loop_body(j, acc):
        sc_block(j, 2 * j, 0, last=False)
        sc_block(j, 2 * j + 1, 1, last=True)
        return acc

    lax.fori_loop(0, N_SEQ_CHUNKS // 2, loop_body, 0)

    wait_o(xb3)


_sc_add = functools.partial(
    pl.kernel,
    out_type=jax.ShapeDtypeStruct((BATCH * SEQ_LEN, D_MODEL), jnp.float32),
    mesh=plsc.VectorSubcoreMesh(core_axis_name="c", subcore_axis_name="s"),
    scratch_types=[
        pltpu.VMEM((CHUNK, D_MODEL), jnp.float32),
        pltpu.VMEM((CHUNK, D_MODEL), jnp.float32),
        pltpu.VMEM((CHUNK, D_MODEL), jnp.float32),
        pltpu.VMEM((CHUNK, D_MODEL), jnp.float32),
        pltpu.VMEM((CHUNK, D_MODEL), jnp.float32),
        pltpu.VMEM((CHUNK, D_MODEL), jnp.float32),
        pltpu.SemaphoreType.DMA,
        pltpu.SemaphoreType.DMA,
        pltpu.SemaphoreType.DMA,
    ],
    compiler_params=pltpu.CompilerParams(use_tc_tiling_on_sc=True),
)(_sc_body)


@jax.jit
def kernel(x, pos_table):
    x2 = x.reshape(BATCH * SEQ_LEN, D_MODEL)
    out = _sc_add(x2, pos_table)
    return out.reshape(x.shape)
